# Initial kernel scaffold; baseline (speedup 1.0000x reference)
import functools

import jax
import jax.numpy as jnp
from jax import lax
from jax.experimental import pallas as pl
from jax.experimental.pallas import tpu as pltpu
from jax.experimental.pallas import tpu_sc as plsc

NEG_SLOPE = 0.01
NC = 2
NS = 16
NW = NC * NS
CHUNK = 128
LANES = 16


def _leaky(v):
    return jnp.where(v >= 0, v, NEG_SLOPE * v)


def _matmul_body(x_ref, wt_ref, h_ref):
    h = jnp.dot(x_ref[...], wt_ref[...], preferred_element_type=jnp.float32)
    h_ref[...] = _leaky(h)


def _combine_body(p_ref, o_ref):
    o_ref[...] = _leaky(p_ref[0] + p_ref[1])


def _make_aggregate(n_nodes, d, chunks_per_tile):
    rows_per_tile = n_nodes // NS
    zcopies = rows_per_tile // CHUNK
    zrem = rows_per_tile - zcopies * CHUNK

    mesh = plsc.VectorSubcoreMesh(
        core_axis_name="c", subcore_axis_name="s", num_cores=NC, num_subcores=NS
    )

    @functools.partial(
        pl.kernel,
        out_type=jax.ShapeDtypeStruct((NC, n_nodes, d), jnp.float32),
        mesh=mesh,
        scratch_types=[
            pltpu.VMEM((CHUNK,), jnp.int32),
            pltpu.VMEM((CHUNK,), jnp.int32),
            pltpu.VMEM((CHUNK,), jnp.float32),
            pltpu.VMEM((CHUNK, d), jnp.float32),
            pltpu.VMEM_SHARED((n_nodes, d), jnp.float32),
            pltpu.SemaphoreType.DMA,
        ],
    )
    def aggregate(h_hbm, row_hbm, col_hbm, w_hbm, out_hbm,
                  col_v, row_v, w_v, rows_v, acc, sem):
        c = lax.axis_index("c")
        s = lax.axis_index("s")
        wid = s * NC + c

        def zero_body(e, _):
            for dd in range(d // LANES):
                rows_v[e, pl.ds(dd * LANES, LANES)] = jnp.zeros(
                    (LANES,), jnp.float32)
            return 0
        lax.fori_loop(0, CHUNK, zero_body, 0, unroll=False)

        zbase = s * rows_per_tile
        for j in range(zcopies):
            pltpu.sync_copy(rows_v, acc.at[pl.ds(zbase + j * CHUNK, CHUNK)])
        if zrem:
            pltpu.sync_copy(rows_v.at[pl.ds(0, zrem)],
                            acc.at[pl.ds(zbase + zcopies * CHUNK, zrem)])
        plsc.subcore_barrier()

        edges_per_tile = chunks_per_tile * CHUNK
        ebase = wid * edges_per_tile

        def window(j, _):
            base = ebase + j * CHUNK
            pltpu.sync_copy(col_hbm.at[pl.ds(base, CHUNK)], col_v)
            pltpu.sync_copy(row_hbm.at[pl.ds(base, CHUNK)], row_v)
            pltpu.sync_copy(w_hbm.at[pl.ds(base, CHUNK)], w_v)
            pltpu.async_copy(h_hbm.at[col_v], rows_v, sem).wait()

            def scale(e, _):
                w = w_v[e]
                for dd in range(d // LANES):
                    sl = pl.ds(dd * LANES, LANES)
                    rows_v[e, sl] = rows_v[e, sl] * w
                return 0
            lax.fori_loop(0, CHUNK, scale, 0, unroll=False)

            pltpu.sync_copy(rows_v, acc.at[row_v], add=True)
            return 0
        lax.fori_loop(0, chunks_per_tile, window, 0, unroll=False)

        plsc.subcore_barrier()

        pltpu.sync_copy(acc.at[pl.ds(s * rows_per_tile, rows_per_tile)],
                        out_hbm.at[c, pl.ds(s * rows_per_tile, rows_per_tile)])

    return aggregate


def kernel(input, edge_index, edge_weight, W):
    n, d_in = input.shape
    d_out = W.shape[0]
    e = edge_index.shape[1]

    row = edge_index[0].astype(jnp.int32)
    col = edge_index[1].astype(jnp.int32)
    w = edge_weight.astype(jnp.float32)

    tile_quantum = NW * CHUNK
    e_pad = ((e + tile_quantum - 1) // tile_quantum) * tile_quantum
    pad = e_pad - e
    if pad:
        pad_idx = jnp.arange(pad, dtype=jnp.int32) % n
        row = jnp.concatenate([row, pad_idx])
        col = jnp.concatenate([col, pad_idx])
        w = jnp.concatenate([w, jnp.zeros((pad,), jnp.float32)])
    chunks_per_tile = e_pad // tile_quantum

    h = pl.pallas_call(
        _matmul_body,
        out_shape=jax.ShapeDtypeStruct((n, d_out), jnp.float32),
    )(input, W.T)

    partials = _make_aggregate(n, d_out, chunks_per_tile)(h, row, col, w)

    out = pl.pallas_call(
        _combine_body,
        out_shape=jax.ShapeDtypeStruct((n, d_out), jnp.float32),
    )(partials)
    return out


# speedup vs baseline: 5.6182x; 5.6182x over previous
import functools

import jax
import jax.numpy as jnp
from jax import lax
from jax.experimental import pallas as pl
from jax.experimental.pallas import tpu as pltpu
from jax.experimental.pallas import tpu_sc as plsc

NEG_SLOPE = 0.01
NC = 2
NS = 16
NW = NC * NS
CHUNK = 128
LANES = 16


def _leaky(v):
    return jnp.where(v >= 0, v, NEG_SLOPE * v)


def _matmul_body(x_ref, wt_ref, h_ref):
    h = jnp.dot(x_ref[...], wt_ref[...], preferred_element_type=jnp.float32)
    h_ref[...] = _leaky(h)


def _combine_body(p_ref, o_ref):
    o_ref[...] = _leaky(p_ref[0] + p_ref[1])


def _make_aggregate(n_nodes, d, chunks_per_tile):
    stripe = (n_nodes // NS) // 8 * 8
    tail = n_nodes - stripe * NS
    zcopies = stripe // CHUNK
    zrem = stripe - zcopies * CHUNK

    mesh = plsc.VectorSubcoreMesh(
        core_axis_name="c", subcore_axis_name="s", num_cores=NC, num_subcores=NS
    )

    @functools.partial(
        pl.kernel,
        out_type=jax.ShapeDtypeStruct((NC, n_nodes, d), jnp.float32),
        mesh=mesh,
        scratch_types=[
            pltpu.VMEM((CHUNK,), jnp.int32),
            pltpu.VMEM((CHUNK,), jnp.int32),
            pltpu.VMEM((CHUNK,), jnp.float32),
            pltpu.VMEM((CHUNK, d), jnp.float32),
            pltpu.VMEM_SHARED((n_nodes, d), jnp.float32),
            pltpu.SemaphoreType.DMA,
        ],
    )
    def aggregate(h_hbm, row_hbm, col_hbm, w_hbm, out_hbm,
                  col_v, row_v, w_v, rows_v, acc, sem):
        c = lax.axis_index("c")
        s = lax.axis_index("s")
        wid = s * NC + c

        def zero_body(e, _):
            for dd in range(d // LANES):
                rows_v[e, pl.ds(dd * LANES, LANES)] = jnp.zeros(
                    (LANES,), jnp.float32)
            return 0
        lax.fori_loop(jnp.int32(0), jnp.int32(CHUNK), zero_body, 0,
                      unroll=False)

        zbase = pl.multiple_of(s * stripe, 8)
        for j in range(zcopies):
            pltpu.sync_copy(rows_v, acc.at[pl.ds(zbase + j * CHUNK, CHUNK)])
        if zrem:
            pltpu.sync_copy(rows_v.at[pl.ds(0, zrem)],
                            acc.at[pl.ds(zbase + zcopies * CHUNK, zrem)])
        if tail:
            @pl.when(s == NS - 1)
            def _zero_tail():
                pltpu.sync_copy(rows_v.at[pl.ds(0, tail)],
                                acc.at[pl.ds(stripe * NS, tail)])
        plsc.subcore_barrier()

        edges_per_tile = chunks_per_tile * CHUNK
        ebase = wid * edges_per_tile

        def window(j, _):
            base = ebase + j * CHUNK
            pltpu.sync_copy(col_hbm.at[pl.ds(base, CHUNK)], col_v)
            pltpu.sync_copy(row_hbm.at[pl.ds(base, CHUNK)], row_v)
            pltpu.sync_copy(w_hbm.at[pl.ds(base, CHUNK)], w_v)
            pltpu.async_copy(h_hbm.at[col_v], rows_v, sem).wait()

            def scale(g, _):
                wvec = w_v[pl.ds(g * LANES, LANES)]
                for l in range(LANES):
                    e = g * LANES + l
                    w = wvec[l]
                    for dd in range(d // LANES):
                        sl = pl.ds(dd * LANES, LANES)
                        rows_v[e, sl] = rows_v[e, sl] * w
                return 0
            lax.fori_loop(jnp.int32(0), jnp.int32(CHUNK // LANES), scale, 0,
                          unroll=False)

            pltpu.sync_copy(rows_v, acc.at[row_v], add=True)
            return 0
        lax.fori_loop(jnp.int32(0), jnp.int32(chunks_per_tile), window, 0,
                      unroll=False)

        plsc.subcore_barrier()

        wbase = pl.multiple_of(s * stripe, 8)
        pltpu.sync_copy(acc.at[pl.ds(wbase, stripe)],
                        out_hbm.at[c, pl.ds(wbase, stripe)])
        if tail:
            @pl.when(s == NS - 1)
            def _write_tail():
                pltpu.sync_copy(acc.at[pl.ds(stripe * NS, tail)],
                                out_hbm.at[c, pl.ds(stripe * NS, tail)])

    return aggregate


def kernel(input, edge_index, edge_weight, W):
    n, d_in = input.shape
    d_out = W.shape[0]
    e = edge_index.shape[1]

    row = edge_index[0].astype(jnp.int32)
    col = edge_index[1].astype(jnp.int32)
    w = edge_weight.astype(jnp.float32)

    tile_quantum = NW * CHUNK
    e_pad = ((e + tile_quantum - 1) // tile_quantum) * tile_quantum
    pad = e_pad - e
    if pad:
        pad_idx = jnp.arange(pad, dtype=jnp.int32) % n
        row = jnp.concatenate([row, pad_idx])
        col = jnp.concatenate([col, pad_idx])
        w = jnp.concatenate([w, jnp.zeros((pad,), jnp.float32)])
    chunks_per_tile = e_pad // tile_quantum

    h = pl.pallas_call(
        _matmul_body,
        out_shape=jax.ShapeDtypeStruct((n, d_out), jnp.float32),
    )(input, W.T)

    partials = _make_aggregate(n, d_out, chunks_per_tile)(h, row, col, w)

    out = pl.pallas_call(
        _combine_body,
        out_shape=jax.ShapeDtypeStruct((n, d_out), jnp.float32),
    )(partials)
    return out


# speedup vs baseline: 9.2696x; 1.6499x over previous
import functools

import jax
import jax.numpy as jnp
from jax import lax
from jax.experimental import pallas as pl
from jax.experimental.pallas import tpu as pltpu
from jax.experimental.pallas import tpu_sc as plsc

NEG_SLOPE = 0.01
NC = 2
NS = 16
NW = NC * NS
CHUNK = 128
LANES = 16


def _leaky(v):
    return jnp.where(v >= 0, v, NEG_SLOPE * v)


def _matmul_body(x_ref, wt_ref, h_ref):
    h = jnp.dot(x_ref[...], wt_ref[...], preferred_element_type=jnp.float32)
    h_ref[...] = _leaky(h)


def _combine_body(p_ref, o_ref):
    o_ref[...] = _leaky(p_ref[0] + p_ref[1])


def _make_aggregate(n_nodes, d, chunks_per_tile):
    stripe = (n_nodes // NS) // 8 * 8
    tail = n_nodes - stripe * NS
    zcopies = stripe // CHUNK
    zrem = stripe - zcopies * CHUNK
    npass = 2
    wpp = chunks_per_tile // npass
    half = wpp // 2

    mesh = plsc.VectorSubcoreMesh(
        core_axis_name="c", subcore_axis_name="s", num_cores=NC, num_subcores=NS
    )

    @functools.partial(
        pl.kernel,
        out_type=jax.ShapeDtypeStruct((NC, n_nodes, d), jnp.float32),
        mesh=mesh,
        scratch_types=[
            pltpu.VMEM((chunks_per_tile // 2, CHUNK), jnp.int32),
            pltpu.VMEM((chunks_per_tile // 2, CHUNK), jnp.int32),
            pltpu.VMEM((chunks_per_tile // 2, CHUNK), jnp.float32),
            pltpu.VMEM((CHUNK, d), jnp.float32),
            pltpu.VMEM((CHUNK, d), jnp.float32),
            pltpu.VMEM_SHARED((n_nodes, d), jnp.float32),
            pltpu.SemaphoreType.DMA,
            pltpu.SemaphoreType.DMA,
            pltpu.SemaphoreType.DMA,
            pltpu.SemaphoreType.DMA,
        ],
    )
    def aggregate(h_hbm, row_hbm, col_hbm, w_hbm, out_hbm,
                  col_v, row_v, w_v, rows0, rows1, acc,
                  gsem0, gsem1, ssem0, ssem1):
        c = lax.axis_index("c")
        s = lax.axis_index("s")
        wid = s * NC + c
        slots = ((rows0, gsem0, ssem0), (rows1, gsem1, ssem1))

        def zero_body(e, _):
            for dd in range(d // LANES):
                rows0[e, pl.ds(dd * LANES, LANES)] = jnp.zeros(
                    (LANES,), jnp.float32)
            return 0
        lax.fori_loop(jnp.int32(0), jnp.int32(CHUNK), zero_body, 0,
                      unroll=False)

        zbase = pl.multiple_of(s * stripe, 8)
        for j in range(zcopies):
            pltpu.sync_copy(rows0, acc.at[pl.ds(zbase + j * CHUNK, CHUNK)])
        if zrem:
            pltpu.sync_copy(rows0.at[pl.ds(0, zrem)],
                            acc.at[pl.ds(zbase + zcopies * CHUNK, zrem)])
        if tail:
            @pl.when(s == NS - 1)
            def _zero_tail():
                pltpu.sync_copy(rows0.at[pl.ds(0, tail)],
                                acc.at[pl.ds(stripe * NS, tail)])
        plsc.subcore_barrier()

        def start_gather(j, slot):
            rbuf, gsem, _ = slots[slot]
            pltpu.async_copy(h_hbm.at[col_v.at[j]], rbuf, gsem)

        def wait_gather(j, slot):
            rbuf, gsem, _ = slots[slot]
            pltpu.make_async_copy(h_hbm.at[col_v.at[j]], rbuf, gsem).wait()

        def start_scatter(j, slot):
            rbuf, _, ssem = slots[slot]
            pltpu.async_copy(rbuf, acc.at[row_v.at[j]], ssem, add=True)

        def wait_scatter(j, slot):
            rbuf, _, ssem = slots[slot]
            pltpu.make_async_copy(rbuf, acc.at[row_v.at[j]], ssem).wait()

        def scale(j, slot):
            rbuf = slots[slot][0]

            def group(g, _):
                wvec = w_v[j, pl.ds(g * LANES, LANES)]
                for l in range(LANES):
                    e = g * LANES + l
                    w = wvec[l]
                    for dd in range(d // LANES):
                        sl = pl.ds(dd * LANES, LANES)
                        rbuf[e, sl] = rbuf[e, sl] * w
                return 0
            lax.fori_loop(jnp.int32(0), jnp.int32(CHUNK // LANES), group, 0,
                          unroll=False)

        def pipeline(j2, _):
            for b in range(2):
                j = j2 * 2 + b
                wait_gather(j, b)
                scale(j, b)
                start_scatter(j, b)
                if b == 0:
                    @pl.when(j2 > 0)
                    def _wait_prev():
                        wait_scatter(j - 1, 1)
                    start_gather(j + 1, 1)
                else:
                    wait_scatter(j - 1, 0)

                    @pl.when(j2 < half - 1)
                    def _next_gather():
                        start_gather(j + 1, 0)
            return 0

        for p in range(npass):
            psl = pl.ds(p * wpp, wpp)
            pltpu.sync_copy(col_hbm.at[wid, psl], col_v)
            pltpu.sync_copy(row_hbm.at[wid, psl], row_v)
            pltpu.sync_copy(w_hbm.at[wid, psl], w_v)
            start_gather(jnp.int32(0), 0)
            lax.fori_loop(jnp.int32(0), jnp.int32(half), pipeline, 0,
                          unroll=False)
            wait_scatter(jnp.int32(wpp - 1), 1)
        plsc.subcore_barrier()

        wbase = pl.multiple_of(s * stripe, 8)
        pltpu.sync_copy(acc.at[pl.ds(wbase, stripe)],
                        out_hbm.at[c, pl.ds(wbase, stripe)])
        if tail:
            @pl.when(s == NS - 1)
            def _write_tail():
                pltpu.sync_copy(acc.at[pl.ds(stripe * NS, tail)],
                                out_hbm.at[c, pl.ds(stripe * NS, tail)])

    return aggregate


def kernel(input, edge_index, edge_weight, W):
    n, d_in = input.shape
    d_out = W.shape[0]
    e = edge_index.shape[1]

    row = edge_index[0].astype(jnp.int32)
    col = edge_index[1].astype(jnp.int32)
    w = edge_weight.astype(jnp.float32)

    tile_quantum = NW * CHUNK * 4
    e_pad = ((e + tile_quantum - 1) // tile_quantum) * tile_quantum
    pad = e_pad - e
    if pad:
        pad_idx = jnp.arange(pad, dtype=jnp.int32) % n
        row = jnp.concatenate([row, pad_idx])
        col = jnp.concatenate([col, pad_idx])
        w = jnp.concatenate([w, jnp.zeros((pad,), jnp.float32)])
    chunks_per_tile = e_pad // (NW * CHUNK)

    row = row.reshape(NW, chunks_per_tile, CHUNK)
    col = col.reshape(NW, chunks_per_tile, CHUNK)
    w = w.reshape(NW, chunks_per_tile, CHUNK)

    h = pl.pallas_call(
        _matmul_body,
        out_shape=jax.ShapeDtypeStruct((n, d_out), jnp.float32),
    )(input, W.T)

    partials = _make_aggregate(n, d_out, chunks_per_tile)(h, row, col, w)

    out = pl.pallas_call(
        _combine_body,
        out_shape=jax.ShapeDtypeStruct((n, d_out), jnp.float32),
    )(partials)
    return out


# speedup vs baseline: 11.6072x; 1.2522x over previous
import functools

import jax
import jax.numpy as jnp
from jax import lax
from jax.experimental import pallas as pl
from jax.experimental.pallas import tpu as pltpu
from jax.experimental.pallas import tpu_sc as plsc

NEG_SLOPE = 0.01
NC = 2
NS = 16
NW = NC * NS
CHUNK = 128
LANES = 16
NSLOT = 3


def _leaky(v):
    return jnp.where(v >= 0, v, NEG_SLOPE * v)


def _matmul_body(x_ref, wt_ref, h_ref):
    h = jnp.dot(x_ref[...], wt_ref[...], preferred_element_type=jnp.float32)
    h_ref[...] = _leaky(h)


def _combine_body(p_ref, o_ref):
    o_ref[...] = _leaky(p_ref[0] + p_ref[1])


def _make_aggregate(n_nodes, d, nwin):
    stripe = (n_nodes // NS) // 8 * 8
    tail = n_nodes - stripe * NS
    zcopies = stripe // CHUNK
    zrem = stripe - zcopies * CHUNK
    assert nwin % NSLOT == 0 and nwin // NSLOT >= 2

    mesh = plsc.VectorSubcoreMesh(
        core_axis_name="c", subcore_axis_name="s", num_cores=NC, num_subcores=NS
    )

    @functools.partial(
        pl.kernel,
        out_type=jax.ShapeDtypeStruct((NC, n_nodes, d), jnp.float32),
        mesh=mesh,
        scratch_types=[
            [pltpu.VMEM((CHUNK, d), jnp.float32) for _ in range(NSLOT)],
            [pltpu.VMEM((CHUNK,), jnp.int32) for _ in range(NSLOT)],
            [pltpu.VMEM((CHUNK,), jnp.int32) for _ in range(NSLOT)],
            [pltpu.VMEM((CHUNK,), jnp.float32) for _ in range(NSLOT)],
            pltpu.VMEM_SHARED((n_nodes, d), jnp.float32),
            [pltpu.SemaphoreType.DMA for _ in range(5 * NSLOT)],
        ],
    )
    def aggregate(h_hbm, row_hbm, col_hbm, w_hbm, out_hbm,
                  rows_s, col_s, row_s, w_s, acc, sems):
        c = lax.axis_index("c")
        s = lax.axis_index("s")
        wid = s * NC + c
        gsem = sems[0:NSLOT]
        ssem = sems[NSLOT:2 * NSLOT]
        csem = sems[2 * NSLOT:3 * NSLOT]
        rsem = sems[3 * NSLOT:4 * NSLOT]
        wsem = sems[4 * NSLOT:5 * NSLOT]
        ebase = wid * (nwin * CHUNK)

        def zero_body(e, _):
            for dd in range(d // LANES):
                rows_s[0][e, pl.ds(dd * LANES, LANES)] = jnp.zeros(
                    (LANES,), jnp.float32)
            return 0
        lax.fori_loop(jnp.int32(0), jnp.int32(CHUNK), zero_body, 0,
                      unroll=False)

        zbase = pl.multiple_of(s * stripe, 8)
        for k in range(zcopies):
            pltpu.sync_copy(rows_s[0], acc.at[pl.ds(zbase + k * CHUNK, CHUNK)])
        if zrem:
            pltpu.sync_copy(rows_s[0].at[pl.ds(0, zrem)],
                            acc.at[pl.ds(zbase + zcopies * CHUNK, zrem)])
        if tail:
            @pl.when(s == NS - 1)
            def _zero_tail():
                pltpu.sync_copy(rows_s[0].at[pl.ds(0, tail)],
                                acc.at[pl.ds(stripe * NS, tail)])
        plsc.subcore_barrier()

        def start_col(j, b):
            pltpu.async_copy(col_hbm.at[pl.ds(ebase + j * CHUNK, CHUNK)],
                             col_s[b], csem[b])

        def wait_col(j, b):
            pltpu.make_async_copy(
                col_hbm.at[pl.ds(ebase + j * CHUNK, CHUNK)],
                col_s[b], csem[b]).wait()

        def start_rw(j, b):
            pltpu.async_copy(row_hbm.at[pl.ds(ebase + j * CHUNK, CHUNK)],
                             row_s[b], rsem[b])
            pltpu.async_copy(w_hbm.at[pl.ds(ebase + j * CHUNK, CHUNK)],
                             w_s[b], wsem[b])

        def wait_rw(j, b):
            pltpu.make_async_copy(
                row_hbm.at[pl.ds(ebase + j * CHUNK, CHUNK)],
                row_s[b], rsem[b]).wait()
            pltpu.make_async_copy(
                w_hbm.at[pl.ds(ebase + j * CHUNK, CHUNK)],
                w_s[b], wsem[b]).wait()

        def start_gather(b):
            pltpu.async_copy(h_hbm.at[col_s[b]], rows_s[b], gsem[b])

        def wait_gather(b):
            pltpu.make_async_copy(h_hbm.at[col_s[b]], rows_s[b],
                                  gsem[b]).wait()

        def start_scatter(b):
            pltpu.async_copy(rows_s[b], acc.at[row_s[b]], ssem[b], add=True)

        def wait_scatter(b):
            pltpu.make_async_copy(rows_s[b], acc.at[row_s[b]], ssem[b]).wait()

        def scale(b):
            rbuf = rows_s[b]

            def group(g, _):
                wvec = w_s[b][pl.ds(g * LANES, LANES)]
                for l in range(LANES):
                    e = g * LANES + l
                    wv = wvec[l]
                    for dd in range(d // LANES):
                        sl = pl.ds(dd * LANES, LANES)
                        rbuf[e, sl] = rbuf[e, sl] * wv
                return 0
            lax.fori_loop(jnp.int32(0), jnp.int32(CHUNK // LANES), group, 0,
                          unroll=False)

        z = jnp.int32(0)
        start_col(z, 0)
        start_col(z + 1, 1)
        start_rw(z, 0)
        start_rw(z + 1, 1)
        wait_col(z, 0)
        start_gather(0)

        def block(i, _):
            for b in range(NSLOT):
                j = i * NSLOT + b
                bn = (b + 1) % NSLOT
                bn2 = (b + 2) % NSLOT

                @pl.when(j >= 2)
                def _free_next_slot():
                    wait_scatter(bn)

                @pl.when(jnp.logical_and(j >= 1, j + 1 < nwin))
                def _prefetch_rw():
                    start_rw(j + 1, bn)

                @pl.when(j + 2 < nwin)
                def _prefetch_col():
                    start_col(j + 2, bn2)

                wait_gather(b)

                @pl.when(j + 1 < nwin)
                def _next_gather():
                    wait_col(j + 1, bn)
                    start_gather(bn)

                wait_rw(j, b)
                scale(b)
                start_scatter(b)
            return 0
        lax.fori_loop(jnp.int32(0), jnp.int32(nwin // NSLOT), block, 0,
                      unroll=False)

        wait_scatter((nwin - 2) % NSLOT)
        wait_scatter((nwin - 1) % NSLOT)
        plsc.subcore_barrier()

        wbase = pl.multiple_of(s * stripe, 8)
        pltpu.sync_copy(acc.at[pl.ds(wbase, stripe)],
                        out_hbm.at[c, pl.ds(wbase, stripe)])
        if tail:
            @pl.when(s == NS - 1)
            def _write_tail():
                pltpu.sync_copy(acc.at[pl.ds(stripe * NS, tail)],
                                out_hbm.at[c, pl.ds(stripe * NS, tail)])

    return aggregate


def kernel(input, edge_index, edge_weight, W):
    n, d_in = input.shape
    d_out = W.shape[0]
    e = edge_index.shape[1]

    row = edge_index[0].astype(jnp.int32)
    col = edge_index[1].astype(jnp.int32)
    w = edge_weight.astype(jnp.float32)

    tile_quantum = NW * CHUNK * NSLOT
    e_pad = ((e + tile_quantum - 1) // tile_quantum) * tile_quantum
    pad = e_pad - e
    if pad:
        pad_idx = jnp.arange(pad, dtype=jnp.int32) % n
        row = jnp.concatenate([row, pad_idx])
        col = jnp.concatenate([col, pad_idx])
        w = jnp.concatenate([w, jnp.zeros((pad,), jnp.float32)])
    nwin = e_pad // (NW * CHUNK)

    h = pl.pallas_call(
        _matmul_body,
        out_shape=jax.ShapeDtypeStruct((n, d_out), jnp.float32),
    )(input, W.T)

    partials = _make_aggregate(n, d_out, nwin)(h, row, col, w)

    out = pl.pallas_call(
        _combine_body,
        out_shape=jax.ShapeDtypeStruct((n, d_out), jnp.float32),
    )(partials)
    return out
